# baseline (speedup 1.0000x reference)
import functools

import jax
import jax.numpy as jnp
from jax import lax
from jax.experimental import pallas as pl
from jax.experimental.pallas import tpu as pltpu
from jax.experimental.pallas import tpu_sc as plsc

B = 16384
D = 32
NROWS = 1000000
LINE = 128
RPL = LINE // D
NLINES = 253952
L = 16
NC, NS = 2, 16
NW = NC * NS
BPW = B // NW
NCHUNK = 2
CH = BPW // NCHUNK
CGROUPS = CH // L

TC_K = 4096
TC_GRID = NLINES // TC_K

_MESH = plsc.VectorSubcoreMesh(core_axis_name="c", subcore_axis_name="s")


def _slot(ids):
    one = jnp.int32(1)
    zero = jnp.int32(0)
    return (jnp.where(ids >= NLINES, one, zero)
            + jnp.where(ids >= 2 * NLINES, one, zero)
            + jnp.where(ids >= 3 * NLINES, one, zero))


def _repack_body(x0, x1, x2, x3, out_ref):
    out_ref[...] = jnp.concatenate(
        [xc[...] for xc in (x0, x1, x2, x3)], axis=0).T


def _repack(tableT):
    last_block = NROWS // TC_K

    def _imap(c, i):
        return (0, jnp.minimum(TC_GRID * c + i, last_block))

    in_specs = [
        pl.BlockSpec((D, TC_K), functools.partial(_imap, c))
        for c in range(RPL)
    ]
    return pl.pallas_call(
        _repack_body,
        grid=(TC_GRID,),
        in_specs=in_specs,
        out_specs=pl.BlockSpec((TC_K, LINE), lambda i: (i, 0)),
        out_shape=jax.ShapeDtypeStruct((NLINES, LINE), jnp.float32),
    )(tableT, tableT, tableT, tableT)


@functools.partial(
    pl.kernel,
    out_type=jax.ShapeDtypeStruct((B,), jnp.float32),
    mesh=_MESH,
    scratch_types=[
        pltpu.VMEM((BPW,), jnp.int32),
        pltpu.VMEM((BPW,), jnp.int32),
        pltpu.VMEM((CH,), jnp.int32),
        pltpu.VMEM((CH,), jnp.int32),
        pltpu.VMEM((CH, LINE), jnp.float32),
        pltpu.VMEM((CH, LINE), jnp.float32),
        pltpu.VMEM((BPW,), jnp.float32),
        pltpu.VMEM((BPW,), jnp.float32),
        pltpu.VMEM((BPW,), jnp.float32),
        pltpu.SemaphoreType.DMA,
        pltpu.SemaphoreType.DMA,
    ],
    compiler_params=pltpu.CompilerParams(needs_layout_passes=False,
                                         use_tc_tiling_on_sc=False),
)
def _mf_sc(uids_hbm, iids_hbm, ulines_hbm, vlines_hbm, ubias_hbm, ibias_hbm,
           out_hbm, idx_u, idx_i, lidx_u, lidx_i, ulines, vlines,
           ub_v, ib_v, out_v, sem, bsem):
    wid = lax.axis_index("s") * NC + lax.axis_index("c")
    base = wid * BPW

    pltpu.sync_copy(uids_hbm.at[pl.ds(base, BPW)], idx_u)
    pltpu.sync_copy(iids_hbm.at[pl.ds(base, BPW)], idx_i)

    cpb1 = pltpu.async_copy(ubias_hbm.at[idx_u], ub_v, bsem)
    cpb2 = pltpu.async_copy(ibias_hbm.at[idx_i], ib_v, bsem)

    iota16 = lax.iota(jnp.int32, L)

    for c in range(NCHUNK):
        cbase = c * CH

        def lines(g, carry):
            ids_u = idx_u[pl.ds(cbase + g * L, L)]
            ids_i = idx_i[pl.ds(cbase + g * L, L)]
            lidx_u[pl.ds(g * L, L)] = ids_u - _slot(ids_u) * NLINES
            lidx_i[pl.ds(g * L, L)] = ids_i - _slot(ids_i) * NLINES
            return carry

        lax.fori_loop(0, CGROUPS, lines, 0)

        cp1 = pltpu.async_copy(ulines_hbm.at[lidx_u], ulines, sem)
        cp2 = pltpu.async_copy(vlines_hbm.at[lidx_i], vlines, sem)
        cp1.wait()
        cp2.wait()
        if c == 0:
            cpb1.wait()
            cpb2.wait()

        def group(g, carry):
            gb = cbase + g * L
            ids_u = idx_u[pl.ds(gb, L)]
            ids_i = idx_i[pl.ds(gb, L)]
            sub_u = _slot(ids_u) * D
            sub_i = _slot(ids_i) * D
            evec = g * L + iota16
            acc = ub_v[pl.ds(gb, L)] + ib_v[pl.ds(gb, L)]
            for d in range(D):
                acc = acc + plsc.load_gather(ulines, [evec, sub_u + d]) * \
                    plsc.load_gather(vlines, [evec, sub_i + d])
            out_v[pl.ds(gb, L)] = acc
            return carry

        lax.fori_loop(0, CGROUPS, group, 0)

    pltpu.sync_copy(out_v, out_hbm.at[pl.ds(base, BPW)])


def kernel(user_ids, item_ids, user_emb, item_emb, user_bias, item_bias):
    ulines = _repack(user_emb.T)
    vlines = _repack(item_emb.T)
    return _mf_sc(user_ids.astype(jnp.int32), item_ids.astype(jnp.int32),
                  ulines, vlines,
                  user_bias.reshape(-1), item_bias.reshape(-1))


# speedup vs baseline: 1.1038x; 1.1038x over previous
import functools

import jax
import jax.numpy as jnp
from jax import lax
from jax.experimental import pallas as pl
from jax.experimental.pallas import tpu as pltpu
from jax.experimental.pallas import tpu_sc as plsc

B = 16384
D = 32
NROWS = 1000000
LINE = 128
RPL = LINE // D
NLINES = 262144
L = 16
NC, NS = 2, 16
NW = NC * NS
BPW = B // NW
NCHUNK = 2
CH = BPW // NCHUNK
CGROUPS = CH // L

TC_K = 16384
TC_GRID = NLINES // TC_K

_MESH = plsc.VectorSubcoreMesh(core_axis_name="c", subcore_axis_name="s")


def _slot(ids):
    one = jnp.int32(1)
    zero = jnp.int32(0)
    return (jnp.where(ids >= NLINES, one, zero)
            + jnp.where(ids >= 2 * NLINES, one, zero)
            + jnp.where(ids >= 3 * NLINES, one, zero))


def _repack_body(x0, x1, x2, x3, out_ref):
    out_ref[...] = jnp.concatenate(
        [xc[...] for xc in (x0, x1, x2, x3)], axis=0).T


def _repack(tableT):
    last_block = NROWS // TC_K

    def _imap(c, i):
        return (0, jnp.minimum(TC_GRID * c + i, last_block))

    in_specs = [
        pl.BlockSpec((D, TC_K), functools.partial(_imap, c))
        for c in range(RPL)
    ]
    return pl.pallas_call(
        _repack_body,
        grid=(TC_GRID,),
        in_specs=in_specs,
        out_specs=pl.BlockSpec((TC_K, LINE), lambda i: (i, 0)),
        out_shape=jax.ShapeDtypeStruct((NLINES, LINE), jnp.float32),
    )(tableT, tableT, tableT, tableT)


@functools.partial(
    pl.kernel,
    out_type=jax.ShapeDtypeStruct((B,), jnp.float32),
    mesh=_MESH,
    scratch_types=[
        pltpu.VMEM((BPW,), jnp.int32),
        pltpu.VMEM((BPW,), jnp.int32),
        pltpu.VMEM((CH,), jnp.int32),
        pltpu.VMEM((CH,), jnp.int32),
        pltpu.VMEM((CH, LINE), jnp.float32),
        pltpu.VMEM((CH, LINE), jnp.float32),
        pltpu.VMEM((BPW,), jnp.float32),
        pltpu.VMEM((BPW,), jnp.float32),
        pltpu.VMEM((BPW,), jnp.float32),
        pltpu.SemaphoreType.DMA,
        pltpu.SemaphoreType.DMA,
    ],
    compiler_params=pltpu.CompilerParams(needs_layout_passes=False,
                                         use_tc_tiling_on_sc=False),
)
def _mf_sc(uids_hbm, iids_hbm, ulines_hbm, vlines_hbm, ubias_hbm, ibias_hbm,
           out_hbm, idx_u, idx_i, lidx_u, lidx_i, ulines, vlines,
           ub_v, ib_v, out_v, sem, bsem):
    wid = lax.axis_index("s") * NC + lax.axis_index("c")
    base = wid * BPW

    pltpu.sync_copy(uids_hbm.at[pl.ds(base, BPW)], idx_u)
    pltpu.sync_copy(iids_hbm.at[pl.ds(base, BPW)], idx_i)

    cpb1 = pltpu.async_copy(ubias_hbm.at[idx_u], ub_v, bsem)
    cpb2 = pltpu.async_copy(ibias_hbm.at[idx_i], ib_v, bsem)

    iota16 = lax.iota(jnp.int32, L)

    for c in range(NCHUNK):
        cbase = c * CH

        def lines(g, carry):
            ids_u = idx_u[pl.ds(cbase + g * L, L)]
            ids_i = idx_i[pl.ds(cbase + g * L, L)]
            lidx_u[pl.ds(g * L, L)] = ids_u - _slot(ids_u) * NLINES
            lidx_i[pl.ds(g * L, L)] = ids_i - _slot(ids_i) * NLINES
            return carry

        lax.fori_loop(0, CGROUPS, lines, 0)

        cp1 = pltpu.async_copy(ulines_hbm.at[lidx_u], ulines, sem)
        cp2 = pltpu.async_copy(vlines_hbm.at[lidx_i], vlines, sem)
        cp1.wait()
        cp2.wait()
        if c == 0:
            cpb1.wait()
            cpb2.wait()

        def group(g, carry):
            gb = cbase + g * L
            ids_u = idx_u[pl.ds(gb, L)]
            ids_i = idx_i[pl.ds(gb, L)]
            sub_u = _slot(ids_u) * D
            sub_i = _slot(ids_i) * D
            evec = g * L + iota16
            acc = ub_v[pl.ds(gb, L)] + ib_v[pl.ds(gb, L)]
            for d in range(D):
                acc = acc + plsc.load_gather(ulines, [evec, sub_u + d]) * \
                    plsc.load_gather(vlines, [evec, sub_i + d])
            out_v[pl.ds(gb, L)] = acc
            return carry

        lax.fori_loop(0, CGROUPS, group, 0)

    pltpu.sync_copy(out_v, out_hbm.at[pl.ds(base, BPW)])


def kernel(user_ids, item_ids, user_emb, item_emb, user_bias, item_bias):
    ulines = _repack(user_emb.T)
    vlines = _repack(item_emb.T)
    return _mf_sc(user_ids.astype(jnp.int32), item_ids.astype(jnp.int32),
                  ulines, vlines,
                  user_bias.reshape(-1), item_bias.reshape(-1))


# speedup vs baseline: 1.1307x; 1.0244x over previous
import functools

import jax
import jax.numpy as jnp
from jax import lax
from jax.experimental import pallas as pl
from jax.experimental.pallas import tpu as pltpu
from jax.experimental.pallas import tpu_sc as plsc

B = 16384
D = 32
NROWS = 1000000
LINE = 128
RPL = LINE // D
NLINES = 253952
L = 16
NC, NS = 2, 16
NW = NC * NS
BPW = B // NW
NCHUNK = 2
CH = BPW // NCHUNK
CGROUPS = CH // L

TC_K = 8192
TC_GRID = NLINES // TC_K

_MESH = plsc.VectorSubcoreMesh(core_axis_name="c", subcore_axis_name="s")


def _slot(ids):
    one = jnp.int32(1)
    zero = jnp.int32(0)
    return (jnp.where(ids >= NLINES, one, zero)
            + jnp.where(ids >= 2 * NLINES, one, zero)
            + jnp.where(ids >= 3 * NLINES, one, zero))


def _repack_body(x0, x1, x2, x3, y0, y1, y2, y3, out_u, out_v):
    out_u[...] = jnp.concatenate(
        [xc[...] for xc in (x0, x1, x2, x3)], axis=0).T
    out_v[...] = jnp.concatenate(
        [yc[...] for yc in (y0, y1, y2, y3)], axis=0).T


def _repack(utableT, vtableT):
    last_block = NROWS // TC_K

    def _imap(c, i):
        return (0, jnp.minimum(TC_GRID * c + i, last_block))

    in_specs = [
        pl.BlockSpec((D, TC_K), functools.partial(_imap, c))
        for c in range(RPL)
    ] * 2
    out_spec = pl.BlockSpec((TC_K, LINE), lambda i: (i, 0))
    return pl.pallas_call(
        _repack_body,
        grid=(TC_GRID,),
        in_specs=in_specs,
        out_specs=[out_spec, out_spec],
        out_shape=[jax.ShapeDtypeStruct((NLINES, LINE), jnp.float32)] * 2,
    )(utableT, utableT, utableT, utableT,
      vtableT, vtableT, vtableT, vtableT)


@functools.partial(
    pl.kernel,
    out_type=jax.ShapeDtypeStruct((B,), jnp.float32),
    mesh=_MESH,
    scratch_types=[
        pltpu.VMEM((BPW,), jnp.int32),
        pltpu.VMEM((BPW,), jnp.int32),
        pltpu.VMEM((CH,), jnp.int32),
        pltpu.VMEM((CH,), jnp.int32),
        pltpu.VMEM((CH, LINE), jnp.float32),
        pltpu.VMEM((CH, LINE), jnp.float32),
        pltpu.VMEM((BPW,), jnp.float32),
        pltpu.VMEM((BPW,), jnp.float32),
        pltpu.VMEM((BPW,), jnp.float32),
        pltpu.SemaphoreType.DMA,
        pltpu.SemaphoreType.DMA,
    ],
    compiler_params=pltpu.CompilerParams(needs_layout_passes=False,
                                         use_tc_tiling_on_sc=False),
)
def _mf_sc(uids_hbm, iids_hbm, ulines_hbm, vlines_hbm, ubias_hbm, ibias_hbm,
           out_hbm, idx_u, idx_i, lidx_u, lidx_i, ulines, vlines,
           ub_v, ib_v, out_v, sem, bsem):
    wid = lax.axis_index("s") * NC + lax.axis_index("c")
    base = wid * BPW

    pltpu.sync_copy(uids_hbm.at[pl.ds(base, BPW)], idx_u)
    pltpu.sync_copy(iids_hbm.at[pl.ds(base, BPW)], idx_i)

    cpb1 = pltpu.async_copy(ubias_hbm.at[idx_u], ub_v, bsem)
    cpb2 = pltpu.async_copy(ibias_hbm.at[idx_i], ib_v, bsem)

    iota16 = lax.iota(jnp.int32, L)

    for c in range(NCHUNK):
        cbase = c * CH

        def lines(g, carry):
            ids_u = idx_u[pl.ds(cbase + g * L, L)]
            ids_i = idx_i[pl.ds(cbase + g * L, L)]
            lidx_u[pl.ds(g * L, L)] = ids_u - _slot(ids_u) * NLINES
            lidx_i[pl.ds(g * L, L)] = ids_i - _slot(ids_i) * NLINES
            return carry

        lax.fori_loop(0, CGROUPS, lines, 0)

        cp1 = pltpu.async_copy(ulines_hbm.at[lidx_u], ulines, sem)
        cp2 = pltpu.async_copy(vlines_hbm.at[lidx_i], vlines, sem)
        cp1.wait()
        cp2.wait()
        if c == 0:
            cpb1.wait()
            cpb2.wait()

        def group(g, carry):
            gb = cbase + g * L
            ids_u = idx_u[pl.ds(gb, L)]
            ids_i = idx_i[pl.ds(gb, L)]
            sub_u = _slot(ids_u) * D
            sub_i = _slot(ids_i) * D
            evec = g * L + iota16
            acc = ub_v[pl.ds(gb, L)] + ib_v[pl.ds(gb, L)]
            for d in range(D):
                acc = acc + plsc.load_gather(ulines, [evec, sub_u + d]) * \
                    plsc.load_gather(vlines, [evec, sub_i + d])
            out_v[pl.ds(gb, L)] = acc
            return carry

        lax.fori_loop(0, CGROUPS, group, 0)

    pltpu.sync_copy(out_v, out_hbm.at[pl.ds(base, BPW)])


def kernel(user_ids, item_ids, user_emb, item_emb, user_bias, item_bias):
    ulines, vlines = _repack(user_emb.T, item_emb.T)
    return _mf_sc(user_ids.astype(jnp.int32), item_ids.astype(jnp.int32),
                  ulines, vlines,
                  user_bias.reshape(-1), item_bias.reshape(-1))
